# baseline (speedup 1.0000x reference)
import functools

import jax
import jax.numpy as jnp
from jax.experimental import pallas as pl

_HIDDEN = 4096
_EXPERTS = 64
_TOPK = 8
_SUB = 128


def _router_body(x_ref, w_ref, logits_ref, wts_ref, idx_ref):
    b = x_ref.shape[0]
    srow_f = jax.lax.broadcasted_iota(
        jnp.int32, (_EXPERTS, _SUB), 0).astype(jnp.float32)
    krow = jax.lax.broadcasted_iota(jnp.int32, (_TOPK, _SUB), 0)
    for s in range(b // _SUB):
        rows = pl.ds(s * _SUB, _SUB)
        x = x_ref[rows, :]
        logits_ref[rows, :] = jax.lax.dot_general(
            x, w_ref[...],
            dimension_numbers=(((1,), (1,)), ((), ())),
            preferred_element_type=jnp.float32,
        )
        lt = jax.lax.dot_general(
            w_ref[...], x,
            dimension_numbers=(((1,), (1,)), ((), ())),
            preferred_element_type=jnp.float32,
        )
        work = lt
        vals = jnp.zeros((_TOPK, _SUB), jnp.float32)
        idxs = jnp.zeros((_TOPK, _SUB), jnp.float32)
        for j in range(_TOPK):
            m = jnp.max(work, axis=0, keepdims=True)
            hit = work == m
            vals = jnp.where(krow == j, m, vals)
            imf = jnp.min(jnp.where(hit, srow_f, float(_EXPERTS)),
                          axis=0, keepdims=True)
            idxs = jnp.where(krow == j, imf, idxs)
            work = jnp.where(hit, -jnp.inf, work)
        ex = jnp.exp(vals - jnp.max(vals, axis=0, keepdims=True))
        wts_ref[:, rows] = ex / jnp.sum(ex, axis=0, keepdims=True)
        idx_ref[:, rows] = idxs.astype(jnp.int32)


@functools.partial(jax.jit, static_argnames=("block_t", "interpret"))
def _router(hidden_states, gate_w, block_t=1024, interpret=False):
    tokens = hidden_states.shape[0]
    grid = (tokens // block_t,)
    return pl.pallas_call(
        _router_body,
        grid=grid,
        in_specs=[
            pl.BlockSpec((block_t, _HIDDEN), lambda i: (i, 0)),
            pl.BlockSpec((_EXPERTS, _HIDDEN), lambda i: (0, 0)),
        ],
        out_specs=[
            pl.BlockSpec((block_t, _EXPERTS), lambda i: (i, 0)),
            pl.BlockSpec((_TOPK, block_t), lambda i: (0, i)),
            pl.BlockSpec((_TOPK, block_t), lambda i: (0, i)),
        ],
        out_shape=[
            jax.ShapeDtypeStruct((tokens, _EXPERTS), jnp.float32),
            jax.ShapeDtypeStruct((_TOPK, tokens), jnp.float32),
            jax.ShapeDtypeStruct((_TOPK, tokens), jnp.int32),
        ],
        interpret=interpret,
    )(hidden_states, gate_w)


def kernel(hidden_states, gate_w):
    logits, wts_t, idx_t = _router(hidden_states, gate_w)
    t = hidden_states.shape[0]
    return (jnp.zeros((t, _TOPK), jnp.float32),
            jnp.zeros((t, _TOPK), jnp.int32), logits)


# speedup vs baseline: 1.1333x; 1.1333x over previous
import functools

import jax
import jax.numpy as jnp
from jax.experimental import pallas as pl

_HIDDEN = 4096
_EXPERTS = 64
_TOPK = 8
_SUB = 128


def _router_body(x_ref, w_ref, logits_ref, wts_ref, idx_ref):
    b = x_ref.shape[0]
    srow_f = jax.lax.broadcasted_iota(
        jnp.int32, (_EXPERTS, _SUB), 0).astype(jnp.float32)
    krow = jax.lax.broadcasted_iota(jnp.int32, (_TOPK, _SUB), 0)
    for s in range(b // _SUB):
        rows = pl.ds(s * _SUB, _SUB)
        x = x_ref[rows, :]
        lt = jax.lax.dot_general(
            w_ref[...], x,
            dimension_numbers=(((1,), (1,)), ((), ())),
            preferred_element_type=jnp.float32,
        )
        logits_ref[rows, :] = lt.T
        work = lt
        vals = jnp.zeros((_TOPK, _SUB), jnp.float32)
        idxs = jnp.zeros((_TOPK, _SUB), jnp.float32)
        for j in range(_TOPK):
            m = jnp.max(work, axis=0, keepdims=True)
            hit = work == m
            vals = jnp.where(krow == j, m, vals)
            imf = jnp.min(jnp.where(hit, srow_f, float(_EXPERTS)),
                          axis=0, keepdims=True)
            idxs = jnp.where(krow == j, imf, idxs)
            work = jnp.where(hit, -jnp.inf, work)
        ex = jnp.exp(vals - jnp.max(vals, axis=0, keepdims=True))
        wts_ref[:, rows] = ex / jnp.sum(ex, axis=0, keepdims=True)
        idx_ref[:, rows] = idxs.astype(jnp.int32)


@functools.partial(jax.jit, static_argnames=("block_t", "interpret"))
def _router(hidden_states, gate_w, block_t=1024, interpret=False):
    tokens = hidden_states.shape[0]
    grid = (tokens // block_t,)
    return pl.pallas_call(
        _router_body,
        grid=grid,
        in_specs=[
            pl.BlockSpec((block_t, _HIDDEN), lambda i: (i, 0)),
            pl.BlockSpec((_EXPERTS, _HIDDEN), lambda i: (0, 0)),
        ],
        out_specs=[
            pl.BlockSpec((block_t, _EXPERTS), lambda i: (i, 0)),
            pl.BlockSpec((_TOPK, block_t), lambda i: (0, i)),
            pl.BlockSpec((_TOPK, block_t), lambda i: (0, i)),
        ],
        out_shape=[
            jax.ShapeDtypeStruct((tokens, _EXPERTS), jnp.float32),
            jax.ShapeDtypeStruct((_TOPK, tokens), jnp.float32),
            jax.ShapeDtypeStruct((_TOPK, tokens), jnp.int32),
        ],
        interpret=interpret,
    )(hidden_states, gate_w)


def kernel(hidden_states, gate_w):
    logits, wts_t, idx_t = _router(hidden_states, gate_w)
    return (wts_t.T, idx_t.T, logits)
